# baseline (speedup 1.0000x reference)
import jax
import jax.numpy as jnp
from jax import lax
from jax.experimental import pallas as pl
from jax.experimental.pallas import tpu as pltpu
from jax.experimental.pallas import tpu_sc as plsc

B = 16384
K = 64
L = 16
NC = 2
NS = 16
NW = NC * NS
BPW = B // NW
CHUNK = 128
NCH = BPW // CHUNK
G = CHUNK // L


def _mf_score_body(uh_hbm, ih_hbm, pou_hbm, poi_hbm, U2_hbm, I2_hbm, out_hbm,
                   uhx_v, ihx_v, pou_v, poi_v, urow_v, irow_v, out_v, sem):
    wid = lax.axis_index("s") * NC + lax.axis_index("c")
    base = wid * BPW

    pltpu.sync_copy(uh_hbm.at[wid], uhx_v)
    pltpu.sync_copy(ih_hbm.at[wid], ihx_v)
    pltpu.sync_copy(pou_hbm.at[pl.ds(base, BPW)], pou_v)
    pltpu.sync_copy(poi_hbm.at[pl.ds(base, BPW)], poi_v)

    for c in range(NCH):
        cp_u = pltpu.async_copy(U2_hbm.at[uhx_v.at[c]], urow_v, sem)
        cp_i = pltpu.async_copy(I2_hbm.at[ihx_v.at[c]], irow_v, sem)
        cp_u.wait()
        cp_i.wait()

        @pl.loop(0, G)
        def _(g):
            rows = lax.iota(jnp.int32, L) + g * L
            cu = pou_v[pl.ds(c * CHUNK + g * L, L)]
            ci = poi_v[pl.ds(c * CHUNK + g * L, L)]
            acc = (plsc.load_gather(urow_v, [rows, cu])
                   * plsc.load_gather(irow_v, [rows, ci]))
            for _ in range(1, K):
                cu = cu + 1
                ci = ci + 1
                acc = acc + (plsc.load_gather(urow_v, [rows, cu])
                             * plsc.load_gather(irow_v, [rows, ci]))
            out_v[pl.ds(c * CHUNK + g * L, L)] = acc

    pltpu.sync_copy(out_v, out_hbm.at[pl.ds(base, BPW)])


@jax.jit
def _mf_score(u, i, U, I):
    U2 = U.reshape(U.shape[0] // 2, 2 * K)
    I2 = I.reshape(I.shape[0] // 2, 2 * K)
    uh = (u >> 1).reshape(NW, NCH, CHUNK)
    ih = (i >> 1).reshape(NW, NCH, CHUNK)
    pou = (u & 1) << 6
    poi = (i & 1) << 6

    mesh = plsc.VectorSubcoreMesh(core_axis_name="c", subcore_axis_name="s")
    cp = pltpu.CompilerParams(needs_layout_passes=False)
    run = pl.kernel(
        _mf_score_body,
        out_type=jax.ShapeDtypeStruct((B,), jnp.float32),
        mesh=mesh,
        scratch_types=[
            pltpu.VMEM((NCH, CHUNK), jnp.int32),
            pltpu.VMEM((NCH, CHUNK), jnp.int32),
            pltpu.VMEM((BPW,), jnp.int32),
            pltpu.VMEM((BPW,), jnp.int32),
            pltpu.VMEM((CHUNK, 2 * K), jnp.float32),
            pltpu.VMEM((CHUNK, 2 * K), jnp.float32),
            pltpu.VMEM((BPW,), jnp.float32),
            pltpu.SemaphoreType.DMA,
        ],
        compiler_params=cp,
    )
    return run(uh, ih, pou, poi, U2, I2)


def kernel(u, i, U, I):
    return _mf_score(u, i, U, I)


# speedup vs baseline: 2.0307x; 2.0307x over previous
import jax
import jax.numpy as jnp
from jax import lax
from jax.experimental import pallas as pl
from jax.experimental.pallas import tpu as pltpu
from jax.experimental.pallas import tpu_sc as plsc

B = 16384
K = 64
L = 16
NC = 2
NS = 16
NW = NC * NS
BPW = B // NW
CW = 512
NV = 1000000
TAIL = (NV // CW) * CW
GR = B + L
SENT = jnp.int32(2**30)


def _gather_body(su_hbm, pu_hbm, si_hbm, pi_hbm, UT_hbm, IT_hbm,
                 gu_hbm, gi_hbm,
                 cs_v, ps_v, blk_v, tlb_v, stg_v, sem):
    wid = lax.axis_index("s") * NC + lax.axis_index("c")
    base = wid * BPW
    lane = lax.iota(jnp.int32, L)
    trs = [(lane + q * L) >> 3 for q in range(K // L)]
    srs = [(lane + q * L) & 7 for q in range(K // L)]

    def one_table(tab_hbm, keys_hbm, pos_hbm, dst_hbm):
        pltpu.sync_copy(keys_hbm.at[pl.ds(base, BPW)], cs_v.at[pl.ds(0, BPW)])
        pltpu.sync_copy(pos_hbm.at[pl.ds(base, BPW)], ps_v.at[pl.ds(0, BPW)])
        cs_v[pl.ds(BPW, L)] = jnp.full((L,), SENT, jnp.int32)
        ps_v[pl.ds(BPW, L)] = jnp.zeros((L,), jnp.int32)

        def outer_body(cursor):
            cols0 = cs_v[pl.ds(cursor, L)]
            wstart = pl.multiple_of((cols0[0] >> 9) << 9, CW)

            @pl.when(wstart < TAIL)
            def _():
                pltpu.sync_copy(tab_hbm.at[:, :, pl.ds(wstart, CW)], blk_v)

            @pl.when(wstart >= TAIL)
            def _():
                pltpu.sync_copy(tab_hbm.at[:, :, pl.ds(TAIL, NV - TAIL)],
                                tlb_v)

            wend = wstart + CW
            in_tail = wstart >= TAIL

            def inner_body(cur):
                cols = cs_v[pl.ds(cur, L)]
                pos = ps_v[pl.ds(cur, L)]
                m = cols < wend
                mi = m.astype(jnp.int32)
                n = plsc.all_reduce_population_count(m)[0]

                @pl.when(jnp.logical_not(in_tail))
                def _():
                    for j in range(L):
                        c = cols[j] - wstart
                        mj = jnp.full((L,), mi[j]) == 1
                        for q in range(K // L):
                            vals = plsc.load_gather(
                                blk_v, [trs[q], srs[q], jnp.full((L,), c)],
                                mask=mj)
                            stg_v[j, pl.ds(q * L, L)] = vals

                @pl.when(in_tail)
                def _():
                    for j in range(L):
                        c = cols[j] - wstart
                        mj = jnp.full((L,), mi[j]) == 1
                        for q in range(K // L):
                            vals = plsc.load_gather(
                                tlb_v, [trs[q], srs[q], jnp.full((L,), c)],
                                mask=mj)
                            stg_v[j, pl.ds(q * L, L)] = vals

                idxv = jnp.where(m, pos, B + lane)
                pltpu.sync_copy(stg_v, dst_hbm.at[idxv])
                return cur + n

            def inner_cond(cur):
                cols = cs_v[pl.ds(cur, L)]
                m = cols < wend
                n = plsc.all_reduce_population_count(m)[0]
                return n == L

            cursor = lax.while_loop(inner_cond, inner_body, cursor)
            return inner_body(cursor)

        lax.while_loop(lambda cur: cur < BPW, outer_body, jnp.int32(0))

    one_table(UT_hbm, su_hbm, pu_hbm, gu_hbm)
    one_table(IT_hbm, si_hbm, pi_hbm, gi_hbm)


def _dot_body(gu_hbm, gi_hbm, out_hbm, gu_v, gi_v, out_v, sem):
    wid = lax.axis_index("s") * NC + lax.axis_index("c")
    base = wid * BPW
    lane = lax.iota(jnp.int32, L)
    last = lane == (L - 1)
    CH = 128

    for ch in range(BPW // CH):
        row0 = base + ch * CH
        pltpu.sync_copy(gu_hbm.at[pl.ds(row0, CH), :], gu_v)
        pltpu.sync_copy(gi_hbm.at[pl.ds(row0, CH), :], gi_v)

        @pl.loop(0, CH)
        def _(r):
            acc = gu_v[r, pl.ds(0, L)] * gi_v[r, pl.ds(0, L)]
            for q in range(1, K // L):
                acc = acc + gu_v[r, pl.ds(q * L, L)] * gi_v[r, pl.ds(q * L, L)]
            total = plsc.cumsum(acc)
            plsc.store_scatter(out_v, [jnp.full((L,), ch * CH + r, jnp.int32)],
                               total, mask=last)

    pltpu.sync_copy(out_v, out_hbm.at[pl.ds(base, BPW)])


@jax.jit
def _mf_score(u, i, U, I):
    UT = U.T.reshape(8, 8, NV)
    IT = I.T.reshape(8, 8, NV)
    iota = jnp.arange(B, dtype=jnp.int32)
    su, pu = lax.sort_key_val(u, iota)
    si, pi_ = lax.sort_key_val(i, iota)

    mesh = plsc.VectorSubcoreMesh(core_axis_name="c", subcore_axis_name="s")
    cp = pltpu.CompilerParams(needs_layout_passes=False)

    gather = pl.kernel(
        _gather_body,
        out_type=(jax.ShapeDtypeStruct((GR, 2 * K), jnp.float32),
                  jax.ShapeDtypeStruct((GR, 2 * K), jnp.float32)),
        mesh=mesh,
        scratch_types=[
            pltpu.VMEM((BPW + L,), jnp.int32),
            pltpu.VMEM((BPW + L,), jnp.int32),
            pltpu.VMEM((8, 8, CW), jnp.float32),
            pltpu.VMEM((8, 8, NV - TAIL), jnp.float32),
            pltpu.VMEM((L, 2 * K), jnp.float32),
            pltpu.SemaphoreType.DMA,
        ],
        compiler_params=cp,
    )
    gu, gi = gather(su, pu, si, pi_, UT, IT)

    dot = pl.kernel(
        _dot_body,
        out_type=jax.ShapeDtypeStruct((B,), jnp.float32),
        mesh=mesh,
        scratch_types=[
            pltpu.VMEM((128, 2 * K), jnp.float32),
            pltpu.VMEM((128, 2 * K), jnp.float32),
            pltpu.VMEM((BPW,), jnp.float32),
            pltpu.SemaphoreType.DMA,
        ],
        compiler_params=cp,
    )
    return dot(gu, gi)


def kernel(u, i, U, I):
    return _mf_score(u, i, U, I)
